# baseline (speedup 1.0000x reference)
import functools

import jax
import jax.numpy as jnp
from jax import lax
from jax.experimental import pallas as pl
from jax.experimental.pallas import tpu as pltpu
from jax.experimental.pallas import tpu_sc as plsc

NC = 2
NS = 16
NW = NC * NS
L = 16
C = 128


def _round_up(x, m):
    return (x + m - 1) // m * m


def _tc_z_body(h_ref, w_ref, a_ref, z_ref, esed_ref):
    zb = jnp.dot(h_ref[...], w_ref[0], preferred_element_type=jnp.float32)
    z_ref[...] = zb[None]
    esb = jnp.dot(zb, a_ref[0].T, preferred_element_type=jnp.float32)
    esed_ref[...] = esb.T[None]


def _make_tc_z(Np, DIN, DOUT, H, BN):
    grid = (Np // BN, H)
    return pl.pallas_call(
        _tc_z_body,
        grid=grid,
        in_specs=[
            pl.BlockSpec((BN, DIN), lambda i, hh: (i, 0)),
            pl.BlockSpec((1, DIN, DOUT), lambda i, hh: (hh, 0, 0)),
            pl.BlockSpec((1, 2, DOUT), lambda i, hh: (hh, 0, 0)),
        ],
        out_specs=[
            pl.BlockSpec((1, BN, DOUT), lambda i, hh: (hh, i, 0)),
            pl.BlockSpec((1, 2, BN), lambda i, hh: (hh, 0, i)),
        ],
        out_shape=[
            jax.ShapeDtypeStruct((H, Np, DOUT), jnp.float32),
            jax.ShapeDtypeStruct((H, 2, Np), jnp.float32),
        ],
    )


CC = 64
NB = 4


def _make_sc_gat(Np, DOUT, H, E, EPTA, EPTB):
    TA = EPTA // CC // NB
    TB = EPTB // CC // NB
    rows_per_tile = Np // NS
    nzero = rows_per_tile // CC

    mesh = plsc.VectorSubcoreMesh(
        core_axis_name="c", subcore_axis_name="s", num_cores=NC,
        num_subcores=NS)

    buf_types = []
    for _ in range(NB):
        buf_types += [
            pltpu.VMEM((CC,), jnp.float32),
            pltpu.VMEM((CC,), jnp.int32),
            pltpu.VMEM((CC,), jnp.int32),
            pltpu.VMEM((CC,), jnp.int32),
            pltpu.VMEM((CC,), jnp.int32),
            pltpu.VMEM((CC,), jnp.float32),
            pltpu.VMEM((CC,), jnp.float32),
            pltpu.VMEM((CC,), jnp.int32),
            pltpu.VMEM((CC,), jnp.int32),
            pltpu.VMEM((CC, DOUT), jnp.float32),
        ]

    @functools.partial(
        pl.kernel,
        out_type=[
            jax.ShapeDtypeStruct((NC, H, Np, DOUT), jnp.float32),
            jax.ShapeDtypeStruct((NC, H, Np), jnp.float32),
        ],
        mesh=mesh,
        scratch_types=[
            *buf_types,
            pltpu.VMEM((rows_per_tile,), jnp.float32),
            pltpu.VMEM_SHARED((Np, DOUT), jnp.float32),
            pltpu.VMEM_SHARED((Np,), jnp.float32),
            *([pltpu.SemaphoreType.DMA] * (3 * NB)),
        ],
    )
    def sc_gat(zflat, esedflat, src_hbm, dst_hbm, num_out, s_out, *rest):
        nfields = 10
        bufs = [rest[b * nfields:(b + 1) * nfields] for b in range(NB)]
        ex_b = [bufs[b][0] for b in range(NB)]
        gidx = [bufs[b][1] for b in range(NB)]
        didx = [bufs[b][2] for b in range(NB)]
        esidx = [bufs[b][3] for b in range(NB)]
        edidx = [bufs[b][4] for b in range(NB)]
        es_b = [bufs[b][5] for b in range(NB)]
        ed_b = [bufs[b][6] for b in range(NB)]
        src_ch = [bufs[b][7] for b in range(NB)]
        dst_ch = [bufs[b][8] for b in range(NB)]
        rows = [bufs[b][9] for b in range(NB)]
        zs_buf = rest[NB * nfields]
        num_sh = rest[NB * nfields + 1]
        s_sh = rest[NB * nfields + 2]
        gsem = list(rest[NB * nfields + 3:NB * nfields + 3 + NB])
        ssem = list(rest[NB * nfields + 3 + NB:NB * nfields + 3 + 2 * NB])
        lsem = list(rest[NB * nfields + 3 + 2 * NB:NB * nfields + 3 + 3 * NB])

        cid = lax.axis_index("c")
        sid = lax.axis_index("s")
        is_a = cid == 1
        rank_a = sid
        rank_b = sid
        base = jnp.where(is_a, rank_a * EPTA,
                         NS * EPTA + rank_b * EPTB)
        T = jnp.where(is_a, TA, TB)

        zeros16 = jnp.zeros((L,), jnp.float32)

        def zs(i, _):
            zs_buf[pl.ds(i * L, L)] = zeros16
            return 0

        lax.fori_loop(0, rows_per_tile // L, zs, 0)

        def head_body(hh, _):
            hNp = hh * Np
            esbase = (2 * hh) * Np
            edbase = (2 * hh + 1) * Np

            def fire_lin(b, c):
                off = base + c * CC
                pltpu.async_copy(src_hbm.at[pl.ds(off, CC)], src_ch[b],
                                 lsem[b])
                pltpu.async_copy(dst_hbm.at[pl.ds(off, CC)], dst_ch[b],
                                 lsem[b])

            def drain_lin(b):
                pltpu.make_async_copy(
                    src_hbm.at[pl.ds(0, CC)], src_ch[b], lsem[b]).wait()
                pltpu.make_async_copy(
                    dst_hbm.at[pl.ds(0, CC)], dst_ch[b], lsem[b]).wait()

            def idx_fire(b, c):
                for g in range(CC // L):
                    gsl = pl.ds(g * L, L)
                    s16 = src_ch[b][gsl]
                    d16 = dst_ch[b][gsl]
                    gidx[b][gsl] = s16 + hNp
                    didx[b][gsl] = d16
                    esidx[b][gsl] = s16 + esbase
                    edidx[b][gsl] = d16 + edbase
                pltpu.async_copy(esedflat.at[esidx[b]], es_b[b], gsem[b])
                pltpu.async_copy(esedflat.at[edidx[b]], ed_b[b], gsem[b])
                pltpu.async_copy(zflat.at[gidx[b]], rows[b], gsem[b])

            def drain_g(b):
                pltpu.make_async_copy(
                    esedflat.at[pl.ds(0, CC)], es_b[b], gsem[b]).wait()
                pltpu.make_async_copy(
                    esedflat.at[pl.ds(0, CC)], ed_b[b], gsem[b]).wait()
                pltpu.make_async_copy(
                    zflat.at[pl.ds(0, CC)], rows[b], gsem[b]).wait()

            def compute_ex(b, c):
                off = c * CC
                for g in range(CC // L):
                    gsl = pl.ds(g * L, L)
                    e = es_b[b][gsl] + ed_b[b][gsl]
                    e = jnp.maximum(e, e * jnp.float32(0.01))
                    ex = jnp.exp(e)
                    eid = base + off + g * L + lax.iota(jnp.int32, L)
                    ex = jnp.where(eid < E, ex, jnp.float32(0.0))
                    ex_b[b][gsl] = ex

            def scale(b):
                def row_body(gi, _):
                    r0 = gi * L
                    ex16 = ex_b[b][pl.ds(r0, L)]
                    for ll in range(L):
                        exr = ex16[ll]
                        for j in range(DOUT // L):
                            sl = pl.ds(j * L, L)
                            rows[b][r0 + ll, sl] = rows[b][r0 + ll, sl] * exr
                    return 0

                lax.fori_loop(0, CC // L, row_body, 0)

            def fire_scat(b):
                pltpu.async_copy(ex_b[b], s_sh.at[didx[b]], ssem[b], add=True)
                pltpu.async_copy(rows[b], num_sh.at[didx[b]], ssem[b],
                                 add=True)

            def drain_s(b):
                pltpu.make_async_copy(
                    esedflat.at[pl.ds(0, CC)], ex_b[b], ssem[b]).wait()
                pltpu.make_async_copy(
                    zflat.at[pl.ds(0, CC)], rows[b], ssem[b]).wait()

            def zrow(r, _):
                for j in range(DOUT // L):
                    rows[0][r, pl.ds(j * L, L)] = zeros16
                return 0

            with jax.named_scope("acc_zero"):
                lax.fori_loop(0, CC, zrow, 0)
                for j in range(nzero):
                    pltpu.sync_copy(
                        rows[0], num_sh.at[pl.ds(sid * rows_per_tile + j * CC,
                                                 CC)])
                pltpu.sync_copy(zs_buf, s_sh.at[pl.ds(sid * rows_per_tile,
                                                      rows_per_tile)])
                plsc.subcore_barrier()

            fire_lin(0, 0)
            fire_lin(1, 1)
            fire_lin(2, 2)
            drain_lin(0)
            idx_fire(0, 0)
            drain_lin(1)
            idx_fire(1, 1)

            def quad(t, _):
                for s in range(NB):
                    b = s
                    c = NB * t + s
                    b3 = (s + 3) % NB
                    if s == 0:
                        fire_lin(b3, c + 3)
                    else:
                        @pl.when(t < T - 1)
                        def _():
                            fire_lin(b3, c + 3)
                    b2 = (s + 2) % NB
                    if s < 2:
                        @pl.when(t > 0)
                        def _():
                            drain_s(b2)

                        drain_lin(b2)
                        idx_fire(b2, c + 2)
                    else:
                        @pl.when(t < T - 1)
                        def _():
                            drain_s(b2)
                            drain_lin(b2)
                            idx_fire(b2, c + 2)
                    drain_g(b)
                    compute_ex(b, c)
                    scale(b)
                    fire_scat(b)
                return 0

            with jax.named_scope("edge_pipe"):
                lax.fori_loop(0, T, quad, 0)
                for b in range(NB):
                    drain_s(b)
                plsc.subcore_barrier()

            with jax.named_scope("copyout"):
                for j in range(nzero):
                    r0 = sid * rows_per_tile + j * CC
                    pltpu.sync_copy(num_sh.at[pl.ds(r0, CC)],
                                    num_out.at[cid, hh, pl.ds(r0, CC)])
                pltpu.sync_copy(
                    s_sh.at[pl.ds(sid * rows_per_tile, rows_per_tile)],
                    s_out.at[cid, hh,
                             pl.ds(sid * rows_per_tile, rows_per_tile)])
                plsc.subcore_barrier()
            return 0

        lax.fori_loop(0, H, head_body, 0)

    return sc_gat


def _tc_combine_body(num_ref, s_ref, out_ref):
    nsum = num_ref[0] + num_ref[1]
    ssum = s_ref[0] + s_ref[1]
    denom = jnp.where(ssum > 0, ssum, jnp.float32(1.0)).T
    out = nsum / denom[:, :, None]
    h, bn, dout = out.shape
    out_ref[...] = out.transpose(1, 0, 2).reshape(bn, h * dout)


def _make_tc_combine(Np, DOUT, H, BN):
    return pl.pallas_call(
        _tc_combine_body,
        grid=(Np // BN,),
        in_specs=[
            pl.BlockSpec((NC, H, BN, DOUT), lambda i: (0, 0, i, 0)),
            pl.BlockSpec((NC, BN, H), lambda i: (0, i, 0)),
        ],
        out_specs=pl.BlockSpec((BN, H * DOUT), lambda i: (i, 0)),
        out_shape=jax.ShapeDtypeStruct((Np, H * DOUT), jnp.float32),
    )


def kernel(h, edge_index, W, A):
    N, DIN = h.shape
    H, DOUT, _ = W.shape
    E = edge_index.shape[1]

    Np = _round_up(N, NS * C)
    quad = CC * NB
    pair = _round_up(_round_up(E, NS) // NS, 2 * quad)
    EPTA = _round_up(int(pair * 0.5), quad)
    EPTB = pair - EPTA
    Ep = pair * NS

    hp = jnp.pad(h, ((0, Np - N), (0, 0)))
    wt4 = W.transpose(0, 2, 1)
    a2 = A.reshape(H, 2, DOUT)
    src = jnp.pad(edge_index[0], (0, Ep - E))
    dst = jnp.pad(edge_index[1], (0, Ep - E))

    z4, esed = _make_tc_z(Np, DIN, DOUT, H, BN=512)(hp, wt4, a2)
    zflat = z4.reshape(H * Np, DOUT)

    num_p, s_p = _make_sc_gat(Np, DOUT, H, E, EPTA, EPTB)(
        zflat, esed.reshape(-1), src, dst)
    s_t = s_p.transpose(0, 2, 1)
    out = _make_tc_combine(Np, DOUT, H, BN=512)(num_p, s_t)
    return out[:N]


# speedup vs baseline: 1.1269x; 1.1269x over previous
import functools

import numpy as np

import jax
import jax.numpy as jnp
from jax import lax
from jax.experimental import pallas as pl
from jax.experimental.pallas import tpu as pltpu
from jax.experimental.pallas import tpu_sc as plsc

NC = 2
NS = 16
NW = NC * NS
L = 16
C = 128


def _round_up(x, m):
    return (x + m - 1) // m * m


def _tc_z_body(h_ref, w_ref, a_ref, z_ref, esed_ref):
    zb = jnp.dot(h_ref[...], w_ref[0], preferred_element_type=jnp.float32)
    z_ref[...] = zb.astype(jnp.bfloat16)[None]
    esb = jnp.dot(zb, a_ref[0].T, preferred_element_type=jnp.float32)
    esed_ref[...] = esb.T[None]


def _make_tc_z(Np, DIN, DOUT, H, BN):
    grid = (Np // BN, H)
    return pl.pallas_call(
        _tc_z_body,
        grid=grid,
        in_specs=[
            pl.BlockSpec((BN, DIN), lambda i, hh: (i, 0)),
            pl.BlockSpec((1, DIN, DOUT), lambda i, hh: (hh, 0, 0)),
            pl.BlockSpec((1, 2, DOUT), lambda i, hh: (hh, 0, 0)),
        ],
        out_specs=[
            pl.BlockSpec((1, BN, DOUT), lambda i, hh: (hh, i, 0)),
            pl.BlockSpec((1, 2, BN), lambda i, hh: (hh, 0, i)),
        ],
        out_shape=[
            jax.ShapeDtypeStruct((H, Np, DOUT), jnp.bfloat16),
            jax.ShapeDtypeStruct((H, 2, Np), jnp.float32),
        ],
    )


CC = 64
NB = 4


def _make_sc_gat(Np, DOUT, H, E, EPTA, EPTB):
    TA = EPTA // CC // NB
    TB = EPTB // CC // NB
    rows_per_tile = Np // NS
    nzero = rows_per_tile // CC

    mesh = plsc.VectorSubcoreMesh(
        core_axis_name="c", subcore_axis_name="s", num_cores=NC,
        num_subcores=NS)

    buf_types = []
    for _ in range(NB):
        buf_types += [
            pltpu.VMEM((CC,), jnp.float32),
            pltpu.VMEM((CC,), jnp.int32),
            pltpu.VMEM((CC,), jnp.int32),
            pltpu.VMEM((CC,), jnp.int32),
            pltpu.VMEM((CC,), jnp.int32),
            pltpu.VMEM((CC,), jnp.float32),
            pltpu.VMEM((CC,), jnp.float32),
            pltpu.VMEM((CC,), jnp.int32),
            pltpu.VMEM((CC,), jnp.int32),
            pltpu.VMEM((CC, DOUT // 2), jnp.int32),
        ]
    scaled_types = [pltpu.VMEM((CC, DOUT), jnp.float32) for _ in range(2)]

    @functools.partial(
        pl.kernel,
        out_type=[
            jax.ShapeDtypeStruct((NC, H, Np, DOUT), jnp.float32),
            jax.ShapeDtypeStruct((NC, H, Np), jnp.float32),
        ],
        mesh=mesh,
        compiler_params=pltpu.CompilerParams(needs_layout_passes=False,
                                             use_tc_tiling_on_sc=False),
        scratch_types=[
            *buf_types,
            *scaled_types,
            pltpu.VMEM((rows_per_tile,), jnp.float32),
            pltpu.VMEM_SHARED((Np, DOUT), jnp.float32),
            pltpu.VMEM_SHARED((Np,), jnp.float32),
            *([pltpu.SemaphoreType.DMA] * (2 * NB + 2)),
        ],
    )
    def sc_gat(zflat, esedflat, src_hbm, dst_hbm, num_out, s_out, *rest):
        nfields = 10
        bufs = [rest[b * nfields:(b + 1) * nfields] for b in range(NB)]
        ex_b = [bufs[b][0] for b in range(NB)]
        gidx = [bufs[b][1] for b in range(NB)]
        didx = [bufs[b][2] for b in range(NB)]
        esidx = [bufs[b][3] for b in range(NB)]
        edidx = [bufs[b][4] for b in range(NB)]
        es_b = [bufs[b][5] for b in range(NB)]
        ed_b = [bufs[b][6] for b in range(NB)]
        src_ch = [bufs[b][7] for b in range(NB)]
        dst_ch = [bufs[b][8] for b in range(NB)]
        rows = [bufs[b][9] for b in range(NB)]
        k = NB * nfields
        scaled = [rest[k], rest[k + 1]]
        zs_buf = rest[k + 2]
        num_sh = rest[k + 3]
        s_sh = rest[k + 4]
        gsem = list(rest[k + 5:k + 5 + NB])
        lsem = list(rest[k + 5 + NB:k + 5 + 2 * NB])
        ssem = [rest[k + 5 + 2 * NB], rest[k + 5 + 2 * NB + 1]]

        cid = lax.axis_index("c")
        sid = lax.axis_index("s")
        is_a = cid == 1
        rank_a = sid
        rank_b = sid
        base = jnp.where(is_a, rank_a * EPTA,
                         NS * EPTA + rank_b * EPTB)
        T = jnp.where(is_a, TA, TB)

        zeros16 = jnp.zeros((L,), jnp.float32)

        def zs(i, _):
            zs_buf[pl.ds(i * L, L)] = zeros16
            return 0

        lax.fori_loop(0, rows_per_tile // L, zs, 0)

        def head_body(hh, _):
            hNp = hh * Np
            esbase = (2 * hh) * Np
            edbase = (2 * hh + 1) * Np

            def fire_lin(b, c):
                off = base + c * CC
                pltpu.async_copy(src_hbm.at[pl.ds(off, CC)], src_ch[b],
                                 lsem[b])
                pltpu.async_copy(dst_hbm.at[pl.ds(off, CC)], dst_ch[b],
                                 lsem[b])

            def drain_lin(b):
                pltpu.make_async_copy(
                    src_hbm.at[pl.ds(0, CC)], src_ch[b], lsem[b]).wait()
                pltpu.make_async_copy(
                    dst_hbm.at[pl.ds(0, CC)], dst_ch[b], lsem[b]).wait()

            def idx_fire(b, c):
                for g in range(CC // L):
                    gsl = pl.ds(g * L, L)
                    s16 = src_ch[b][gsl]
                    d16 = dst_ch[b][gsl]
                    gidx[b][gsl] = s16 + hNp
                    didx[b][gsl] = d16
                    esidx[b][gsl] = s16 + esbase
                    edidx[b][gsl] = d16 + edbase
                pltpu.async_copy(esedflat.at[esidx[b]], es_b[b], gsem[b])
                pltpu.async_copy(esedflat.at[edidx[b]], ed_b[b], gsem[b])
                pltpu.async_copy(zflat.at[gidx[b]], rows[b], gsem[b])

            def drain_g(b):
                pltpu.make_async_copy(
                    esedflat.at[pl.ds(0, CC)], es_b[b], gsem[b]).wait()
                pltpu.make_async_copy(
                    esedflat.at[pl.ds(0, CC)], ed_b[b], gsem[b]).wait()
                pltpu.make_async_copy(
                    zflat.at[pl.ds(0, CC)], rows[b], gsem[b]).wait()

            def compute_ex(b, c):
                off = c * CC
                for g in range(CC // L):
                    gsl = pl.ds(g * L, L)
                    e = es_b[b][gsl] + ed_b[b][gsl]
                    e = jnp.maximum(e, e * jnp.float32(0.01))
                    ex = jnp.exp(e)
                    eid = base + off + g * L + lax.iota(jnp.int32, L)
                    ex = jnp.where(eid < E, ex, jnp.float32(0.0))
                    ex_b[b][gsl] = ex

            def scale(b, p):
                def row_body(gi, _):
                    r0 = gi * L
                    ex16 = ex_b[b][pl.ds(r0, L)]
                    for ll in range(L):
                        exr = ex16[ll]
                        r = r0 + ll
                        for j in range(DOUT // (2 * L)):
                            w = rows[b][r, pl.ds(j * L, L)]
                            lo = plsc.bitcast(w << 16, jnp.float32)
                            hi = plsc.bitcast(
                                w & jnp.int32(-65536), jnp.float32)
                            scaled[p][r, pl.ds(j * 2 * L, L)] = lo * exr
                            scaled[p][r, pl.ds(j * 2 * L + L, L)] = hi * exr
                    return 0

                lax.fori_loop(0, CC // L, row_body, 0)

            def fire_scat(b, p):
                pltpu.async_copy(ex_b[b], s_sh.at[didx[b]], ssem[p], add=True)
                pltpu.async_copy(scaled[p], num_sh.at[didx[b]], ssem[p],
                                 add=True)

            def drain_s(p):
                pltpu.make_async_copy(
                    esedflat.at[pl.ds(0, CC)], ex_b[0], ssem[p]).wait()
                pltpu.make_async_copy(
                    num_out.at[0, 0, pl.ds(0, CC)], scaled[p], ssem[p]).wait()

            def zrow(r, _):
                for j in range(DOUT // L):
                    scaled[0][r, pl.ds(j * L, L)] = zeros16
                return 0

            with jax.named_scope("acc_zero"):
                lax.fori_loop(0, CC, zrow, 0)
                for j in range(nzero):
                    pltpu.sync_copy(
                        scaled[0],
                        num_sh.at[pl.ds(sid * rows_per_tile + j * CC, CC)])
                pltpu.sync_copy(zs_buf, s_sh.at[pl.ds(sid * rows_per_tile,
                                                      rows_per_tile)])
                plsc.subcore_barrier()

            fire_lin(0, 0)
            fire_lin(1, 1)
            fire_lin(2, 2)
            drain_lin(0)
            idx_fire(0, 0)
            drain_lin(1)
            idx_fire(1, 1)

            def quad(t, _):
                for s in range(NB):
                    b = s
                    p = s % 2
                    c = NB * t + s
                    if s < 2:
                        @pl.when(t > 0)
                        def _():
                            drain_s(p)
                    else:
                        drain_s(p)
                    b3 = (s + 3) % NB
                    if s == 0:
                        fire_lin(b3, c + 3)
                    else:
                        @pl.when(t < T - 1)
                        def _():
                            fire_lin(b3, c + 3)
                    b2 = (s + 2) % NB
                    if s < 2:
                        drain_lin(b2)
                        idx_fire(b2, c + 2)
                    else:
                        @pl.when(t < T - 1)
                        def _():
                            drain_lin(b2)
                            idx_fire(b2, c + 2)
                    drain_g(b)
                    compute_ex(b, c)
                    scale(b, p)
                    fire_scat(b, p)
                return 0

            with jax.named_scope("edge_pipe"):
                lax.fori_loop(0, T, quad, 0)
                drain_s(0)
                drain_s(1)
                plsc.subcore_barrier()

            with jax.named_scope("copyout"):
                for j in range(nzero):
                    r0 = sid * rows_per_tile + j * CC
                    pltpu.sync_copy(num_sh.at[pl.ds(r0, CC)],
                                    num_out.at[cid, hh, pl.ds(r0, CC)])
                pltpu.sync_copy(
                    s_sh.at[pl.ds(sid * rows_per_tile, rows_per_tile)],
                    s_out.at[cid, hh,
                             pl.ds(sid * rows_per_tile, rows_per_tile)])
                plsc.subcore_barrier()
            return 0

        lax.fori_loop(0, H, head_body, 0)

    return sc_gat


def _tc_combine_body(num_ref, s_ref, out_ref):
    nsum = num_ref[0] + num_ref[1]
    ssum = s_ref[0] + s_ref[1]
    denom = jnp.where(ssum > 0, ssum, jnp.float32(1.0)).T
    out = nsum / denom[:, :, None]
    h, bn, dout = out.shape
    out_ref[...] = out.transpose(1, 0, 2).reshape(bn, h * dout)


def _make_tc_combine(Np, DOUT, H, BN):
    return pl.pallas_call(
        _tc_combine_body,
        grid=(Np // BN,),
        in_specs=[
            pl.BlockSpec((NC, H, BN, DOUT), lambda i: (0, 0, i, 0)),
            pl.BlockSpec((NC, BN, H), lambda i: (0, i, 0)),
        ],
        out_specs=pl.BlockSpec((BN, H * DOUT), lambda i: (i, 0)),
        out_shape=jax.ShapeDtypeStruct((Np, H * DOUT), jnp.float32),
    )


def kernel(h, edge_index, W, A):
    N, DIN = h.shape
    H, DOUT, _ = W.shape
    E = edge_index.shape[1]

    Np = _round_up(N, NS * C)
    quad = CC * NB
    pair = _round_up(_round_up(E, NS) // NS, 2 * quad)
    EPTA = _round_up(int(pair * 0.5), quad)
    EPTB = pair - EPTA
    Ep = pair * NS

    hp = jnp.pad(h, ((0, Np - N), (0, 0)))
    wt4 = W.transpose(0, 2, 1)
    a2 = A.reshape(H, 2, DOUT)
    src = jnp.pad(edge_index[0], (0, Ep - E))
    dst = jnp.pad(edge_index[1], (0, Ep - E))

    z4, esed = _make_tc_z(Np, DIN, DOUT, H, BN=512)(hp, wt4, a2)
    zflat = jax.lax.bitcast_convert_type(
        z4.reshape(H * Np, DOUT // 2, 2), jnp.int32)

    num_p, s_p = _make_sc_gat(Np, DOUT, H, E, EPTA, EPTB)(
        zflat, esed.reshape(-1), src, dst)
    s_t = s_p.transpose(0, 2, 1)
    out = _make_tc_combine(Np, DOUT, H, BN=512)(num_p, s_t)

    idx128 = np.empty(DOUT, dtype=np.int32)
    for s in range(DOUT):
        r = s % (2 * L)
        o = (s // (2 * L)) * 2 * L + 2 * (r % L) + r // L
        idx128[o] = s
    idx = np.concatenate([hh * DOUT + idx128 for hh in range(H)])
    return jnp.take(out, jnp.asarray(idx), axis=1)[:N]
